# baseline (speedup 1.0000x reference)
import functools

import numpy as np

import jax
import jax.numpy as jnp
from jax import lax
from jax.experimental import pallas as pl
from jax.experimental.pallas import tpu as pltpu
from jax.experimental.pallas import tpu_sc as plsc

NC = 2
NS = 16
NW = NC * NS
CH = 128
ZCH = 64
W = 128
CLAMP = 60.0


def _lrelu(a):
  return jnp.where(a >= 0.0, a, 0.2 * a)


def _elu(a):
  return jnp.where(a > 0.0, a, jnp.exp(a) - 1.0)


def _edge_pass(ta, ad, src, dst, zrows, perm, dmask):
  nacc = (ta.shape[0] + NS * ZCH) // (NS * ZCH) * (NS * ZCH)
  e_pad = src.shape[0]
  epw = e_pad // NW
  nchunk = epw // CH
  rpw = nacc // NS
  nh = perm.shape[0] // 16

  mesh = plsc.VectorSubcoreMesh(
      core_axis_name="c", subcore_axis_name="s",
      num_cores=NC, num_subcores=NS)

  @functools.partial(
      pl.kernel,
      out_type=jax.ShapeDtypeStruct((NC, nacc, W), jnp.float32),
      mesh=mesh,
      scratch_types=[
          pltpu.VMEM((CH,), jnp.int32),
          pltpu.VMEM((CH,), jnp.int32),
          pltpu.VMEM((CH, W), jnp.float32),
          pltpu.VMEM((CH, W), jnp.float32),
          pltpu.VMEM((nh * 16,), jnp.int32),
          pltpu.VMEM((16,), jnp.float32),
          pltpu.VMEM_SHARED((nacc, W), jnp.float32),
          pltpu.SemaphoreType.DMA,
          pltpu.SemaphoreType.DMA,
      ])
  def ek(ta_hbm, ad_hbm, src_hbm, dst_hbm, z_hbm, perm_hbm, dm_hbm, out_hbm,
         sidx, didx, rows, adr, pvm, dvm, acc, sem1, sem2):
    c = lax.axis_index("c")
    s = lax.axis_index("s")
    wid = c * NS + s

    pltpu.sync_copy(perm_hbm, pvm)
    pltpu.sync_copy(dm_hbm, dvm)
    dm = dvm[pl.ds(0, 16)]
    pms = [pvm[pl.ds(16 * k, 16)] for k in range(nh)]

    def zacc(j, carry):
      pltpu.sync_copy(z_hbm, acc.at[pl.ds(s * rpw + j * ZCH, ZCH)])
      return carry
    lax.fori_loop(0, rpw // ZCH, zacc, 0)
    plsc.subcore_barrier()

    base = wid * epw

    def chunk(j, carry):
      off = base + j * CH
      pltpu.sync_copy(src_hbm.at[pl.ds(off, CH)], sidx)
      pltpu.sync_copy(dst_hbm.at[pl.ds(off, CH)], didx)
      cp1 = pltpu.async_copy(ta_hbm.at[sidx], rows, sem1)
      cp2 = pltpu.async_copy(ad_hbm.at[didx], adr, sem2)
      cp1.wait()
      cp2.wait()

      @plsc.parallel_loop(0, CH, 1, unroll=8)
      def edge(e):
        a = rows[e, pl.ds(0, 16)] + adr[e, pl.ds(0, 16)]
        a = jnp.minimum(_lrelu(a), CLAMP)
        w = jnp.exp(a)
        rows[e, pl.ds(0, 16)] = w * dm
        for k in range(nh):
          m = w.at[pms[k]].get(mode="promise_in_bounds")
          o = 16 + k * 16
          rows[e, pl.ds(o, 16)] = rows[e, pl.ds(o, 16)] * m

      pltpu.sync_copy(rows, acc.at[didx], add=True)
      return carry
    lax.fori_loop(0, nchunk, chunk, 0)

    plsc.subcore_barrier()

    def dump(j, carry):
      r0 = s * rpw + j * ZCH
      pltpu.sync_copy(acc.at[pl.ds(r0, ZCH)], out_hbm.at[c, pl.ds(r0, ZCH)])
      return carry
    lax.fori_loop(0, rpw // ZCH, dump, 0)

  return ek(ta, ad, src, dst, zrows, perm, dmask)


def _prep1(x, W1, as1, ad1, *, bn, grid):
  n = x.shape[0]

  def body(x_ref, w_ref, as_ref, adr_ref, ta_ref, ad_ref):
    h = jnp.dot(x_ref[...], w_ref[...], preferred_element_type=jnp.float32)
    hr = h.reshape(bn, 8, 8)
    asrc = (hr * as_ref[...][None]).sum(-1)
    adst = (hr * adr_ref[...][None]).sum(-1)
    z8 = jnp.zeros((bn, 8), jnp.float32)
    z48 = jnp.zeros((bn, 48), jnp.float32)
    z112 = jnp.zeros((bn, 112), jnp.float32)
    ta_ref[...] = jnp.concatenate([asrc, z8, h, z48], axis=1)
    ad_ref[...] = jnp.concatenate([adst, z8, z112], axis=1)

  return pl.pallas_call(
      body,
      grid=(grid,),
      in_specs=[
          pl.BlockSpec((bn, 512), lambda i: (i, 0)),
          pl.BlockSpec((512, 64), lambda i: (0, 0)),
          pl.BlockSpec((8, 8), lambda i: (0, 0)),
          pl.BlockSpec((8, 8), lambda i: (0, 0)),
      ],
      out_specs=[
          pl.BlockSpec((bn, W), lambda i: (i, 0)),
          pl.BlockSpec((bn, W), lambda i: (i, 0)),
      ],
      out_shape=[
          jax.ShapeDtypeStruct((n, W), jnp.float32),
          jax.ShapeDtypeStruct((n, W), jnp.float32),
      ],
  )(x, W1, as1, ad1)


def _mid1(p1, ta1, adt1, b1, W2, as2, ad2, *, bn, grid):
  n = ta1.shape[0]

  def body(p_ref, ta_ref, ad_ref, b_ref, w_ref, as_ref, adr_ref,
           t2a_ref, t2b_ref, t2c_ref, ad2_ref):
    num = p_ref[0, :, 16:80] + p_ref[1, :, 16:80]
    den = p_ref[0, :, 0:8] + p_ref[1, :, 0:8]
    asrc = ta_ref[:, 0:8]
    h = ta_ref[:, 16:80]
    adst = ad_ref[:, 0:8]
    ws = jnp.exp(jnp.minimum(_lrelu(asrc + adst), CLAMP))
    den = den + ws
    num = num + (ws[:, :, None] * h.reshape(bn, 8, 8)).reshape(bn, 64)
    o = (num.reshape(bn, 8, 8) / den[:, :, None]).reshape(bn, 64)
    x2 = _elu(o + b_ref[...])
    h2 = jnp.dot(x2, w_ref[...], preferred_element_type=jnp.float32)
    h2r = h2.reshape(bn, 8, 32)
    a2s = (h2r * as_ref[...][None]).sum(-1)
    a2d = (h2r * adr_ref[...][None]).sum(-1)
    z8 = jnp.zeros((bn, 8), jnp.float32)
    z80 = jnp.zeros((bn, 80), jnp.float32)
    z112 = jnp.zeros((bn, 112), jnp.float32)
    t2a_ref[...] = jnp.concatenate([a2s, z8, h2[:, 0:112]], axis=1)
    t2b_ref[...] = jnp.concatenate([a2s, z8, h2[:, 112:224]], axis=1)
    t2c_ref[...] = jnp.concatenate([a2s, z8, h2[:, 224:256], z80], axis=1)
    ad2_ref[...] = jnp.concatenate([a2d, z8, z112], axis=1)

  return pl.pallas_call(
      body,
      grid=(grid,),
      in_specs=[
          pl.BlockSpec((2, bn, W), lambda i: (0, i, 0)),
          pl.BlockSpec((bn, W), lambda i: (i, 0)),
          pl.BlockSpec((bn, W), lambda i: (i, 0)),
          pl.BlockSpec((1, 64), lambda i: (0, 0)),
          pl.BlockSpec((64, 256), lambda i: (0, 0)),
          pl.BlockSpec((8, 32), lambda i: (0, 0)),
          pl.BlockSpec((8, 32), lambda i: (0, 0)),
      ],
      out_specs=[
          pl.BlockSpec((bn, W), lambda i: (i, 0)),
          pl.BlockSpec((bn, W), lambda i: (i, 0)),
          pl.BlockSpec((bn, W), lambda i: (i, 0)),
          pl.BlockSpec((bn, W), lambda i: (i, 0)),
      ],
      out_shape=[
          jax.ShapeDtypeStruct((n, W), jnp.float32),
          jax.ShapeDtypeStruct((n, W), jnp.float32),
          jax.ShapeDtypeStruct((n, W), jnp.float32),
          jax.ShapeDtypeStruct((n, W), jnp.float32),
      ],
  )(p1, ta1, adt1, b1, W2, as2, ad2)


def _mid2(p2a, p2b, p2c, t2a, t2b, t2c, ad2, b2, W3, as3, ad3, *, bn, grid):
  n = t2a.shape[0]

  def body(pa_ref, pb_ref, pc_ref, taa_ref, tab_ref, tac_ref, ad_ref,
           b_ref, w_ref, as_ref, adr_ref, ta3_ref, ad3_ref):
    numa = pa_ref[0, :, 16:128] + pa_ref[1, :, 16:128]
    numb = pb_ref[0, :, 16:128] + pb_ref[1, :, 16:128]
    numc = pc_ref[0, :, 16:48] + pc_ref[1, :, 16:48]
    den = pa_ref[0, :, 0:8] + pa_ref[1, :, 0:8]
    asrc = taa_ref[:, 0:8]
    adst = ad_ref[:, 0:8]
    h2 = jnp.concatenate(
        [taa_ref[:, 16:128], tab_ref[:, 16:128], tac_ref[:, 16:48]], axis=1)
    ws = jnp.exp(jnp.minimum(_lrelu(asrc + adst), CLAMP))
    den = den + ws
    num = jnp.concatenate([numa, numb, numc], axis=1)
    num = num + (ws[:, :, None] * h2.reshape(bn, 8, 32)).reshape(bn, 256)
    o = (num.reshape(bn, 8, 32) / den[:, :, None]).mean(axis=1)
    x3 = _elu(o + b_ref[...])
    h3 = jnp.dot(x3, w_ref[...], preferred_element_type=jnp.float32)
    a3s = (h3 * as_ref[...]).sum(-1, keepdims=True)
    a3d = (h3 * adr_ref[...]).sum(-1, keepdims=True)
    z96 = jnp.zeros((bn, 96), jnp.float32)
    z112 = jnp.zeros((bn, 112), jnp.float32)
    ta3_ref[...] = jnp.concatenate(
        [jnp.broadcast_to(a3s, (bn, 16)), h3, z96], axis=1)
    ad3_ref[...] = jnp.concatenate(
        [jnp.broadcast_to(a3d, (bn, 16)), z112], axis=1)

  return pl.pallas_call(
      body,
      grid=(grid,),
      in_specs=[
          pl.BlockSpec((2, bn, W), lambda i: (0, i, 0)),
          pl.BlockSpec((2, bn, W), lambda i: (0, i, 0)),
          pl.BlockSpec((2, bn, W), lambda i: (0, i, 0)),
          pl.BlockSpec((bn, W), lambda i: (i, 0)),
          pl.BlockSpec((bn, W), lambda i: (i, 0)),
          pl.BlockSpec((bn, W), lambda i: (i, 0)),
          pl.BlockSpec((bn, W), lambda i: (i, 0)),
          pl.BlockSpec((1, 32), lambda i: (0, 0)),
          pl.BlockSpec((32, 16), lambda i: (0, 0)),
          pl.BlockSpec((1, 16), lambda i: (0, 0)),
          pl.BlockSpec((1, 16), lambda i: (0, 0)),
      ],
      out_specs=[
          pl.BlockSpec((bn, W), lambda i: (i, 0)),
          pl.BlockSpec((bn, W), lambda i: (i, 0)),
      ],
      out_shape=[
          jax.ShapeDtypeStruct((n, W), jnp.float32),
          jax.ShapeDtypeStruct((n, W), jnp.float32),
      ],
  )(p2a, p2b, p2c, t2a, t2b, t2c, ad2, b2, W3, as3, ad3)


def _final(p3, ta3, adt3, b3, batch2d, linW, linb, *, n, ng):

  def body(p_ref, ta_ref, ad_ref, b_ref, bat_ref, lw_ref, lb_ref, o_ref):
    num = p_ref[0, :n, 16:32] + p_ref[1, :n, 16:32]
    den = p_ref[0, :n, 0:1] + p_ref[1, :n, 0:1]
    a = ta_ref[:, 0:1] + ad_ref[:, 0:1]
    ws = jnp.exp(jnp.minimum(_lrelu(a), CLAMP))
    den = den + ws
    num = num + ws * ta_ref[:, 16:32]
    o3 = num / den + b_ref[...]
    gids = lax.broadcasted_iota(jnp.int32, (ng, n), 0)
    onehot = (bat_ref[...] == gids).astype(jnp.float32)
    sums = jnp.dot(onehot, o3, preferred_element_type=jnp.float32)
    cnts = onehot.sum(axis=1, keepdims=True)
    pooled = sums / jnp.maximum(cnts, 1.0)
    o_ref[...] = (
        jnp.dot(pooled, lw_ref[...], preferred_element_type=jnp.float32)
        + lb_ref[...])

  return pl.pallas_call(
      body,
      out_shape=jax.ShapeDtypeStruct((ng, 2), jnp.float32),
  )(p3, ta3, adt3, b3, batch2d, linW, linb)


def _perm_l1():
  return np.concatenate(
      [np.repeat([2 * k, 2 * k + 1], 8) for k in range(4)]).astype(np.int32)


def _perm_l2(off, nh):
  return np.concatenate(
      [np.full(16, (off + 16 * k) // 32) for k in range(nh)]).astype(np.int32)


_PERM1 = _perm_l1()
_PERM2A = _perm_l2(0, 7)
_PERM2B = _perm_l2(112, 7)
_PERM2C = _perm_l2(224, 2)
_PERM3 = np.zeros(16, np.int32)
_DM8 = np.concatenate([np.ones(8), np.zeros(8)]).astype(np.float32)
_DM1 = np.concatenate([np.ones(1), np.zeros(15)]).astype(np.float32)
_DM0 = np.zeros(16, np.float32)


def kernel(x, edge_index, batch, W1, att_src1, att_dst1, b1,
           W2, att_src2, att_dst2, b2, W3, att_src3, att_dst3, b3,
           linW, linb):
  n = x.shape[0]
  e = edge_index.shape[1]
  ng = 32

  src = edge_index[0].astype(jnp.int32)
  dst = edge_index[1].astype(jnp.int32)
  nacc = (n + NS * ZCH) // (NS * ZCH) * (NS * ZCH)
  e_pad = -(-e // (NW * CH)) * (NW * CH)
  if e_pad > e:
    npad = e_pad - e
    spread = np.arange(npad, dtype=np.int32) % (nacc - n)
    src = jnp.concatenate([src, jnp.zeros((npad,), jnp.int32)])
    dst = jnp.concatenate([dst, jnp.asarray(spread + n, jnp.int32)])

  bn = n if n <= 2000 else 2000
  grid = n // bn

  as1 = att_src1.reshape(8, 8)
  ad1 = att_dst1.reshape(8, 8)
  as2 = att_src2.reshape(8, 32)
  ad2 = att_dst2.reshape(8, 32)
  as3 = att_src3.reshape(1, 16)
  ad3 = att_dst3.reshape(1, 16)
  b1r = b1.reshape(1, 64)
  b2r = b2.reshape(1, 32)
  b3r = b3.reshape(1, 16)
  lbr = linb.reshape(1, 2)
  batch2d = batch.astype(jnp.int32).reshape(1, n)

  zrows = jnp.zeros((ZCH, W), jnp.float32)
  perm1 = jnp.asarray(_PERM1)
  perm2a = jnp.asarray(_PERM2A)
  perm2b = jnp.asarray(_PERM2B)
  perm2c = jnp.asarray(_PERM2C)
  perm3 = jnp.asarray(_PERM3)
  dm8 = jnp.asarray(_DM8)
  dm1 = jnp.asarray(_DM1)
  dm0 = jnp.asarray(_DM0)

  ta1, adt1 = _prep1(x, W1, as1, ad1, bn=bn, grid=grid)
  p1 = _edge_pass(ta1, adt1, src, dst, zrows, perm1, dm8)
  t2a, t2b, t2c, adt2 = _mid1(p1, ta1, adt1, b1r, W2, as2, ad2,
                              bn=bn, grid=grid)
  p2a = _edge_pass(t2a, adt2, src, dst, zrows, perm2a, dm8)
  p2b = _edge_pass(t2b, adt2, src, dst, zrows, perm2b, dm0)
  p2c = _edge_pass(t2c, adt2, src, dst, zrows, perm2c, dm0)
  ta3, adt3 = _mid2(p2a, p2b, p2c, t2a, t2b, t2c, adt2, b2r, W3, as3, ad3,
                    bn=bn, grid=grid)
  p3 = _edge_pass(ta3, adt3, src, dst, zrows, perm3, dm1)
  return _final(p3, ta3, adt3, b3r, batch2d, linW, lbr, n=n, ng=ng)


# speedup vs baseline: 1.1081x; 1.1081x over previous
import functools

import numpy as np

import jax
import jax.numpy as jnp
from jax import lax
from jax.experimental import pallas as pl
from jax.experimental.pallas import tpu as pltpu
from jax.experimental.pallas import tpu_sc as plsc

NC = 2
NS = 16
NW = NC * NS
CH = 128
CHE = 64
ZCH = 64
W = 128
CLAMP = 60.0


def _lrelu(a):
  return jnp.where(a >= 0.0, a, 0.2 * a)


def _elu(a):
  return jnp.where(a > 0.0, a, jnp.exp(a) - 1.0)


def _edge_pass(ta, ad, src, dst, zrows, perm, dmask):
  nacc = (ta.shape[0] + NS * ZCH) // (NS * ZCH) * (NS * ZCH)
  e_pad = src.shape[0]
  epw = e_pad // NW
  nchunk = epw // CHE
  rpw = nacc // NS
  nh = perm.shape[0] // 16

  mesh = plsc.VectorSubcoreMesh(
      core_axis_name="c", subcore_axis_name="s",
      num_cores=NC, num_subcores=NS)

  @functools.partial(
      pl.kernel,
      out_type=jax.ShapeDtypeStruct((NC, nacc, W), jnp.float32),
      mesh=mesh,
      scratch_types=[
          pltpu.VMEM((CHE,), jnp.int32),
          pltpu.VMEM((CHE,), jnp.int32),
          pltpu.VMEM((CHE,), jnp.int32),
          pltpu.VMEM((CHE,), jnp.int32),
          pltpu.VMEM((CHE, W), jnp.float32),
          pltpu.VMEM((CHE, W), jnp.float32),
          pltpu.VMEM((CHE, W), jnp.float32),
          pltpu.VMEM((CHE, W), jnp.float32),
          pltpu.VMEM((nh * 16,), jnp.int32),
          pltpu.VMEM((16,), jnp.float32),
          pltpu.VMEM_SHARED((nacc, W), jnp.float32),
          pltpu.SemaphoreType.DMA,
          pltpu.SemaphoreType.DMA,
          pltpu.SemaphoreType.DMA,
          pltpu.SemaphoreType.DMA,
          pltpu.SemaphoreType.DMA,
          pltpu.SemaphoreType.DMA,
          pltpu.SemaphoreType.DMA,
          pltpu.SemaphoreType.DMA,
      ])
  def ek(ta_hbm, ad_hbm, src_hbm, dst_hbm, z_hbm, perm_hbm, dm_hbm, out_hbm,
         sidx0, didx0, sidx1, didx1, rows0, adr0, rows1, adr1, pvm, dvm, acc,
         si0, di0, si1, di1, sr0, sa0, sr1, sa1):
    c = lax.axis_index("c")
    s = lax.axis_index("s")
    wid = c * NS + s

    pltpu.sync_copy(perm_hbm, pvm)
    pltpu.sync_copy(dm_hbm, dvm)
    dm = dvm[pl.ds(0, 16)]
    pms = [pvm[pl.ds(16 * k, 16)] for k in range(nh)]

    def zacc(j, carry):
      pltpu.sync_copy(z_hbm, acc.at[pl.ds(s * rpw + j * ZCH, ZCH)])
      return carry
    lax.fori_loop(0, rpw // ZCH, zacc, 0)
    plsc.subcore_barrier()

    base = wid * epw

    def edges(rows, adr):
      @plsc.parallel_loop(0, CHE, 1, unroll=4)
      def edge(e):
        a = rows[e, pl.ds(0, 16)] + adr[e, pl.ds(0, 16)]
        a = jnp.minimum(_lrelu(a), CLAMP)
        w = jnp.exp(a)
        rows[e, pl.ds(0, 16)] = w * dm
        for k in range(nh):
          m = w.at[pms[k]].get(mode="promise_in_bounds")
          o = 16 + k * 16
          rows[e, pl.ds(o, 16)] = rows[e, pl.ds(o, 16)] * m

    def pair(t, carry):
      off0 = base + (2 * t) * CHE
      off1 = off0 + CHE
      ci0 = pltpu.async_copy(src_hbm.at[pl.ds(off0, CHE)], sidx0, si0)
      ci1 = pltpu.async_copy(dst_hbm.at[pl.ds(off0, CHE)], didx0, di0)
      ci2 = pltpu.async_copy(src_hbm.at[pl.ds(off1, CHE)], sidx1, si1)
      ci3 = pltpu.async_copy(dst_hbm.at[pl.ds(off1, CHE)], didx1, di1)
      ci0.wait()
      ci1.wait()
      cp0 = pltpu.async_copy(ta_hbm.at[sidx0], rows0, sr0)
      cp1 = pltpu.async_copy(ad_hbm.at[didx0], adr0, sa0)
      ci2.wait()
      ci3.wait()
      cp2 = pltpu.async_copy(ta_hbm.at[sidx1], rows1, sr1)
      cp3 = pltpu.async_copy(ad_hbm.at[didx1], adr1, sa1)
      cp0.wait()
      cp1.wait()
      edges(rows0, adr0)
      pltpu.sync_copy(rows0, acc.at[didx0], add=True)
      cp2.wait()
      cp3.wait()
      edges(rows1, adr1)
      pltpu.sync_copy(rows1, acc.at[didx1], add=True)
      return carry
    lax.fori_loop(0, nchunk // 2, pair, 0)

    plsc.subcore_barrier()

    def dump(j, carry):
      r0 = s * rpw + j * ZCH
      pltpu.sync_copy(acc.at[pl.ds(r0, ZCH)], out_hbm.at[c, pl.ds(r0, ZCH)])
      return carry
    lax.fori_loop(0, rpw // ZCH, dump, 0)

  return ek(ta, ad, src, dst, zrows, perm, dmask)


def _prep1(x, W1, as1, ad1, *, bn, grid):
  n = x.shape[0]

  def body(x_ref, w_ref, as_ref, adr_ref, ta_ref, ad_ref):
    h = jnp.dot(x_ref[...], w_ref[...], preferred_element_type=jnp.float32)
    hr = h.reshape(bn, 8, 8)
    asrc = (hr * as_ref[...][None]).sum(-1)
    adst = (hr * adr_ref[...][None]).sum(-1)
    z8 = jnp.zeros((bn, 8), jnp.float32)
    z48 = jnp.zeros((bn, 48), jnp.float32)
    z112 = jnp.zeros((bn, 112), jnp.float32)
    ta_ref[...] = jnp.concatenate([asrc, z8, h, z48], axis=1)
    ad_ref[...] = jnp.concatenate([adst, z8, z112], axis=1)

  return pl.pallas_call(
      body,
      grid=(grid,),
      in_specs=[
          pl.BlockSpec((bn, 512), lambda i: (i, 0)),
          pl.BlockSpec((512, 64), lambda i: (0, 0)),
          pl.BlockSpec((8, 8), lambda i: (0, 0)),
          pl.BlockSpec((8, 8), lambda i: (0, 0)),
      ],
      out_specs=[
          pl.BlockSpec((bn, W), lambda i: (i, 0)),
          pl.BlockSpec((bn, W), lambda i: (i, 0)),
      ],
      out_shape=[
          jax.ShapeDtypeStruct((n, W), jnp.float32),
          jax.ShapeDtypeStruct((n, W), jnp.float32),
      ],
  )(x, W1, as1, ad1)


def _mid1(p1, ta1, adt1, b1, W2, as2, ad2, *, bn, grid):
  n = ta1.shape[0]

  def body(p_ref, ta_ref, ad_ref, b_ref, w_ref, as_ref, adr_ref,
           t2a_ref, t2b_ref, t2c_ref, ad2_ref):
    num = p_ref[0, :, 16:80] + p_ref[1, :, 16:80]
    den = p_ref[0, :, 0:8] + p_ref[1, :, 0:8]
    asrc = ta_ref[:, 0:8]
    h = ta_ref[:, 16:80]
    adst = ad_ref[:, 0:8]
    ws = jnp.exp(jnp.minimum(_lrelu(asrc + adst), CLAMP))
    den = den + ws
    num = num + (ws[:, :, None] * h.reshape(bn, 8, 8)).reshape(bn, 64)
    o = (num.reshape(bn, 8, 8) / den[:, :, None]).reshape(bn, 64)
    x2 = _elu(o + b_ref[...])
    h2 = jnp.dot(x2, w_ref[...], preferred_element_type=jnp.float32)
    h2r = h2.reshape(bn, 8, 32)
    a2s = (h2r * as_ref[...][None]).sum(-1)
    a2d = (h2r * adr_ref[...][None]).sum(-1)
    z8 = jnp.zeros((bn, 8), jnp.float32)
    z80 = jnp.zeros((bn, 80), jnp.float32)
    z112 = jnp.zeros((bn, 112), jnp.float32)
    t2a_ref[...] = jnp.concatenate([a2s, z8, h2[:, 0:112]], axis=1)
    t2b_ref[...] = jnp.concatenate([a2s, z8, h2[:, 112:224]], axis=1)
    t2c_ref[...] = jnp.concatenate([a2s, z8, h2[:, 224:256], z80], axis=1)
    ad2_ref[...] = jnp.concatenate([a2d, z8, z112], axis=1)

  return pl.pallas_call(
      body,
      grid=(grid,),
      in_specs=[
          pl.BlockSpec((2, bn, W), lambda i: (0, i, 0)),
          pl.BlockSpec((bn, W), lambda i: (i, 0)),
          pl.BlockSpec((bn, W), lambda i: (i, 0)),
          pl.BlockSpec((1, 64), lambda i: (0, 0)),
          pl.BlockSpec((64, 256), lambda i: (0, 0)),
          pl.BlockSpec((8, 32), lambda i: (0, 0)),
          pl.BlockSpec((8, 32), lambda i: (0, 0)),
      ],
      out_specs=[
          pl.BlockSpec((bn, W), lambda i: (i, 0)),
          pl.BlockSpec((bn, W), lambda i: (i, 0)),
          pl.BlockSpec((bn, W), lambda i: (i, 0)),
          pl.BlockSpec((bn, W), lambda i: (i, 0)),
      ],
      out_shape=[
          jax.ShapeDtypeStruct((n, W), jnp.float32),
          jax.ShapeDtypeStruct((n, W), jnp.float32),
          jax.ShapeDtypeStruct((n, W), jnp.float32),
          jax.ShapeDtypeStruct((n, W), jnp.float32),
      ],
  )(p1, ta1, adt1, b1, W2, as2, ad2)


def _mid2(p2a, p2b, p2c, t2a, t2b, t2c, ad2, b2, W3, as3, ad3, *, bn, grid):
  n = t2a.shape[0]

  def body(pa_ref, pb_ref, pc_ref, taa_ref, tab_ref, tac_ref, ad_ref,
           b_ref, w_ref, as_ref, adr_ref, ta3_ref, ad3_ref):
    numa = pa_ref[0, :, 16:128] + pa_ref[1, :, 16:128]
    numb = pb_ref[0, :, 16:128] + pb_ref[1, :, 16:128]
    numc = pc_ref[0, :, 16:48] + pc_ref[1, :, 16:48]
    den = pa_ref[0, :, 0:8] + pa_ref[1, :, 0:8]
    asrc = taa_ref[:, 0:8]
    adst = ad_ref[:, 0:8]
    h2 = jnp.concatenate(
        [taa_ref[:, 16:128], tab_ref[:, 16:128], tac_ref[:, 16:48]], axis=1)
    ws = jnp.exp(jnp.minimum(_lrelu(asrc + adst), CLAMP))
    den = den + ws
    num = jnp.concatenate([numa, numb, numc], axis=1)
    num = num + (ws[:, :, None] * h2.reshape(bn, 8, 32)).reshape(bn, 256)
    o = (num.reshape(bn, 8, 32) / den[:, :, None]).mean(axis=1)
    x3 = _elu(o + b_ref[...])
    h3 = jnp.dot(x3, w_ref[...], preferred_element_type=jnp.float32)
    a3s = (h3 * as_ref[...]).sum(-1, keepdims=True)
    a3d = (h3 * adr_ref[...]).sum(-1, keepdims=True)
    z96 = jnp.zeros((bn, 96), jnp.float32)
    z112 = jnp.zeros((bn, 112), jnp.float32)
    ta3_ref[...] = jnp.concatenate(
        [jnp.broadcast_to(a3s, (bn, 16)), h3, z96], axis=1)
    ad3_ref[...] = jnp.concatenate(
        [jnp.broadcast_to(a3d, (bn, 16)), z112], axis=1)

  return pl.pallas_call(
      body,
      grid=(grid,),
      in_specs=[
          pl.BlockSpec((2, bn, W), lambda i: (0, i, 0)),
          pl.BlockSpec((2, bn, W), lambda i: (0, i, 0)),
          pl.BlockSpec((2, bn, W), lambda i: (0, i, 0)),
          pl.BlockSpec((bn, W), lambda i: (i, 0)),
          pl.BlockSpec((bn, W), lambda i: (i, 0)),
          pl.BlockSpec((bn, W), lambda i: (i, 0)),
          pl.BlockSpec((bn, W), lambda i: (i, 0)),
          pl.BlockSpec((1, 32), lambda i: (0, 0)),
          pl.BlockSpec((32, 16), lambda i: (0, 0)),
          pl.BlockSpec((1, 16), lambda i: (0, 0)),
          pl.BlockSpec((1, 16), lambda i: (0, 0)),
      ],
      out_specs=[
          pl.BlockSpec((bn, W), lambda i: (i, 0)),
          pl.BlockSpec((bn, W), lambda i: (i, 0)),
      ],
      out_shape=[
          jax.ShapeDtypeStruct((n, W), jnp.float32),
          jax.ShapeDtypeStruct((n, W), jnp.float32),
      ],
  )(p2a, p2b, p2c, t2a, t2b, t2c, ad2, b2, W3, as3, ad3)


def _final(p3, ta3, adt3, b3, batch2d, linW, linb, *, n, ng):

  def body(p_ref, ta_ref, ad_ref, b_ref, bat_ref, lw_ref, lb_ref, o_ref):
    num = p_ref[0, :n, 16:32] + p_ref[1, :n, 16:32]
    den = p_ref[0, :n, 0:1] + p_ref[1, :n, 0:1]
    a = ta_ref[:, 0:1] + ad_ref[:, 0:1]
    ws = jnp.exp(jnp.minimum(_lrelu(a), CLAMP))
    den = den + ws
    num = num + ws * ta_ref[:, 16:32]
    o3 = num / den + b_ref[...]
    gids = lax.broadcasted_iota(jnp.int32, (ng, n), 0)
    onehot = (bat_ref[...] == gids).astype(jnp.float32)
    sums = jnp.dot(onehot, o3, preferred_element_type=jnp.float32)
    cnts = onehot.sum(axis=1, keepdims=True)
    pooled = sums / jnp.maximum(cnts, 1.0)
    o_ref[...] = (
        jnp.dot(pooled, lw_ref[...], preferred_element_type=jnp.float32)
        + lb_ref[...])

  return pl.pallas_call(
      body,
      out_shape=jax.ShapeDtypeStruct((ng, 2), jnp.float32),
  )(p3, ta3, adt3, b3, batch2d, linW, linb)


def _perm_l1():
  return np.concatenate(
      [np.repeat([2 * k, 2 * k + 1], 8) for k in range(4)]).astype(np.int32)


def _perm_l2(off, nh):
  return np.concatenate(
      [np.full(16, (off + 16 * k) // 32) for k in range(nh)]).astype(np.int32)


_PERM1 = _perm_l1()
_PERM2A = _perm_l2(0, 7)
_PERM2B = _perm_l2(112, 7)
_PERM2C = _perm_l2(224, 2)
_PERM3 = np.zeros(16, np.int32)
_DM8 = np.concatenate([np.ones(8), np.zeros(8)]).astype(np.float32)
_DM1 = np.concatenate([np.ones(1), np.zeros(15)]).astype(np.float32)
_DM0 = np.zeros(16, np.float32)


def kernel(x, edge_index, batch, W1, att_src1, att_dst1, b1,
           W2, att_src2, att_dst2, b2, W3, att_src3, att_dst3, b3,
           linW, linb):
  n = x.shape[0]
  e = edge_index.shape[1]
  ng = 32

  src = edge_index[0].astype(jnp.int32)
  dst = edge_index[1].astype(jnp.int32)
  nacc = (n + NS * ZCH) // (NS * ZCH) * (NS * ZCH)
  e_pad = -(-e // (NW * CH)) * (NW * CH)
  if e_pad > e:
    npad = e_pad - e
    spread = np.arange(npad, dtype=np.int32) % (nacc - n)
    src = jnp.concatenate([src, jnp.zeros((npad,), jnp.int32)])
    dst = jnp.concatenate([dst, jnp.asarray(spread + n, jnp.int32)])

  bn = n if n <= 2000 else 2000
  grid = n // bn

  as1 = att_src1.reshape(8, 8)
  ad1 = att_dst1.reshape(8, 8)
  as2 = att_src2.reshape(8, 32)
  ad2 = att_dst2.reshape(8, 32)
  as3 = att_src3.reshape(1, 16)
  ad3 = att_dst3.reshape(1, 16)
  b1r = b1.reshape(1, 64)
  b2r = b2.reshape(1, 32)
  b3r = b3.reshape(1, 16)
  lbr = linb.reshape(1, 2)
  batch2d = batch.astype(jnp.int32).reshape(1, n)

  zrows = jnp.zeros((ZCH, W), jnp.float32)
  perm1 = jnp.asarray(_PERM1)
  perm2a = jnp.asarray(_PERM2A)
  perm2b = jnp.asarray(_PERM2B)
  perm2c = jnp.asarray(_PERM2C)
  perm3 = jnp.asarray(_PERM3)
  dm8 = jnp.asarray(_DM8)
  dm1 = jnp.asarray(_DM1)
  dm0 = jnp.asarray(_DM0)

  ta1, adt1 = _prep1(x, W1, as1, ad1, bn=bn, grid=grid)
  p1 = _edge_pass(ta1, adt1, src, dst, zrows, perm1, dm8)
  t2a, t2b, t2c, adt2 = _mid1(p1, ta1, adt1, b1r, W2, as2, ad2,
                              bn=bn, grid=grid)
  p2a = _edge_pass(t2a, adt2, src, dst, zrows, perm2a, dm8)
  p2b = _edge_pass(t2b, adt2, src, dst, zrows, perm2b, dm0)
  p2c = _edge_pass(t2c, adt2, src, dst, zrows, perm2c, dm0)
  ta3, adt3 = _mid2(p2a, p2b, p2c, t2a, t2b, t2c, adt2, b2r, W3, as3, ad3,
                    bn=bn, grid=grid)
  p3 = _edge_pass(ta3, adt3, src, dst, zrows, perm3, dm1)
  return _final(p3, ta3, adt3, b3r, batch2d, linW, lbr, n=n, ng=ng)


# speedup vs baseline: 1.1259x; 1.0161x over previous
import functools

import numpy as np

import jax
import jax.numpy as jnp
from jax import lax
from jax.experimental import pallas as pl
from jax.experimental.pallas import tpu as pltpu
from jax.experimental.pallas import tpu_sc as plsc

NC = 2
NS = 16
NW = NC * NS
CH = 128
CHE = 64
ZCH = 64
W = 128
CLAMP = 60.0


def _lrelu(a):
  return jnp.where(a >= 0.0, a, 0.2 * a)


def _elu(a):
  return jnp.where(a > 0.0, a, jnp.exp(a) - 1.0)


def _edge_pass(ta, ad, src, dst, zrows, perm, dmask):
  nacc = (ta.shape[0] + NS * ZCH) // (NS * ZCH) * (NS * ZCH)
  e_pad = src.shape[0]
  epw = e_pad // NW
  nchunk = epw // CHE
  rpw = nacc // NS
  nh = perm.shape[0] // 16

  mesh = plsc.VectorSubcoreMesh(
      core_axis_name="c", subcore_axis_name="s",
      num_cores=NC, num_subcores=NS)

  @functools.partial(
      pl.kernel,
      out_type=jax.ShapeDtypeStruct((NC, nacc, W), jnp.float32),
      mesh=mesh,
      scratch_types=[
          pltpu.VMEM((CHE,), jnp.int32),
          pltpu.VMEM((CHE,), jnp.int32),
          pltpu.VMEM((CHE,), jnp.int32),
          pltpu.VMEM((CHE,), jnp.int32),
          pltpu.VMEM((CHE, W), jnp.float32),
          pltpu.VMEM((CHE, W), jnp.float32),
          pltpu.VMEM((CHE, W), jnp.float32),
          pltpu.VMEM((CHE, W), jnp.float32),
          pltpu.VMEM((nh * 16,), jnp.int32),
          pltpu.VMEM((16,), jnp.float32),
          pltpu.VMEM_SHARED((nacc, W), jnp.float32),
          pltpu.SemaphoreType.DMA,
          pltpu.SemaphoreType.DMA,
          pltpu.SemaphoreType.DMA,
          pltpu.SemaphoreType.DMA,
          pltpu.SemaphoreType.DMA,
          pltpu.SemaphoreType.DMA,
          pltpu.SemaphoreType.DMA,
          pltpu.SemaphoreType.DMA,
      ])
  def ek(ta_hbm, ad_hbm, src_hbm, dst_hbm, z_hbm, perm_hbm, dm_hbm, out_hbm,
         sidx0, didx0, sidx1, didx1, rows0, adr0, rows1, adr1, pvm, dvm, acc,
         si0, di0, si1, di1, sr0, sa0, sr1, sa1):
    c = lax.axis_index("c")
    s = lax.axis_index("s")
    wid = c * NS + s

    pltpu.sync_copy(perm_hbm, pvm)
    pltpu.sync_copy(dm_hbm, dvm)
    dm = dvm[pl.ds(0, 16)]
    pms = [pvm[pl.ds(16 * k, 16)] for k in range(nh)]

    def zacc(j, carry):
      pltpu.sync_copy(z_hbm, acc.at[pl.ds(s * rpw + j * ZCH, ZCH)])
      return carry
    lax.fori_loop(0, rpw // ZCH, zacc, 0)
    plsc.subcore_barrier()

    base = wid * epw

    def edges(rows, adr):
      @plsc.parallel_loop(0, CHE, 1, unroll=4)
      def edge(e):
        a = rows[e, pl.ds(0, 16)] + adr[e, pl.ds(0, 16)]
        a = jnp.minimum(_lrelu(a), CLAMP)
        w = jnp.exp(a)
        rows[e, pl.ds(0, 16)] = w * dm
        for k in range(nh):
          m = w.at[pms[k]].get(mode="promise_in_bounds")
          o = 16 + k * 16
          rows[e, pl.ds(o, 16)] = rows[e, pl.ds(o, 16)] * m

    def pair(t, carry):
      off0 = base + (2 * t) * CHE
      off1 = off0 + CHE
      ci0 = pltpu.async_copy(src_hbm.at[pl.ds(off0, CHE)], sidx0, si0)
      ci1 = pltpu.async_copy(dst_hbm.at[pl.ds(off0, CHE)], didx0, di0)
      ci2 = pltpu.async_copy(src_hbm.at[pl.ds(off1, CHE)], sidx1, si1)
      ci3 = pltpu.async_copy(dst_hbm.at[pl.ds(off1, CHE)], didx1, di1)
      ci0.wait()
      ci1.wait()
      cp0 = pltpu.async_copy(ta_hbm.at[sidx0], rows0, sr0)
      cp1 = pltpu.async_copy(ad_hbm.at[didx0], adr0, sa0)
      ci2.wait()
      ci3.wait()
      cp2 = pltpu.async_copy(ta_hbm.at[sidx1], rows1, sr1)
      cp3 = pltpu.async_copy(ad_hbm.at[didx1], adr1, sa1)
      cp0.wait()
      cp1.wait()
      edges(rows0, adr0)
      sc0 = pltpu.async_copy(rows0, acc.at[didx0], si0, add=True)
      cp2.wait()
      cp3.wait()
      edges(rows1, adr1)
      sc1 = pltpu.async_copy(rows1, acc.at[didx1], si1, add=True)
      sc0.wait()
      sc1.wait()
      return carry
    lax.fori_loop(0, nchunk // 2, pair, 0)

    plsc.subcore_barrier()

    def dump(j, carry):
      r0 = s * rpw + j * ZCH
      pltpu.sync_copy(acc.at[pl.ds(r0, ZCH)], out_hbm.at[c, pl.ds(r0, ZCH)])
      return carry
    lax.fori_loop(0, rpw // ZCH, dump, 0)

  return ek(ta, ad, src, dst, zrows, perm, dmask)


def _prep1(x, W1, as1, ad1, *, bn, grid):
  n = x.shape[0]

  def body(x_ref, w_ref, as_ref, adr_ref, ta_ref, ad_ref):
    h = jnp.dot(x_ref[...], w_ref[...], preferred_element_type=jnp.float32)
    hr = h.reshape(bn, 8, 8)
    asrc = (hr * as_ref[...][None]).sum(-1)
    adst = (hr * adr_ref[...][None]).sum(-1)
    z8 = jnp.zeros((bn, 8), jnp.float32)
    z48 = jnp.zeros((bn, 48), jnp.float32)
    z112 = jnp.zeros((bn, 112), jnp.float32)
    ta_ref[...] = jnp.concatenate([asrc, z8, h, z48], axis=1)
    ad_ref[...] = jnp.concatenate([adst, z8, z112], axis=1)

  return pl.pallas_call(
      body,
      grid=(grid,),
      in_specs=[
          pl.BlockSpec((bn, 512), lambda i: (i, 0)),
          pl.BlockSpec((512, 64), lambda i: (0, 0)),
          pl.BlockSpec((8, 8), lambda i: (0, 0)),
          pl.BlockSpec((8, 8), lambda i: (0, 0)),
      ],
      out_specs=[
          pl.BlockSpec((bn, W), lambda i: (i, 0)),
          pl.BlockSpec((bn, W), lambda i: (i, 0)),
      ],
      out_shape=[
          jax.ShapeDtypeStruct((n, W), jnp.float32),
          jax.ShapeDtypeStruct((n, W), jnp.float32),
      ],
  )(x, W1, as1, ad1)


def _mid1(p1, ta1, adt1, b1, W2, as2, ad2, *, bn, grid):
  n = ta1.shape[0]

  def body(p_ref, ta_ref, ad_ref, b_ref, w_ref, as_ref, adr_ref,
           t2a_ref, t2b_ref, t2c_ref, ad2_ref):
    num = p_ref[0, :, 16:80] + p_ref[1, :, 16:80]
    den = p_ref[0, :, 0:8] + p_ref[1, :, 0:8]
    asrc = ta_ref[:, 0:8]
    h = ta_ref[:, 16:80]
    adst = ad_ref[:, 0:8]
    ws = jnp.exp(jnp.minimum(_lrelu(asrc + adst), CLAMP))
    den = den + ws
    num = num + (ws[:, :, None] * h.reshape(bn, 8, 8)).reshape(bn, 64)
    o = (num.reshape(bn, 8, 8) / den[:, :, None]).reshape(bn, 64)
    x2 = _elu(o + b_ref[...])
    h2 = jnp.dot(x2, w_ref[...], preferred_element_type=jnp.float32)
    h2r = h2.reshape(bn, 8, 32)
    a2s = (h2r * as_ref[...][None]).sum(-1)
    a2d = (h2r * adr_ref[...][None]).sum(-1)
    z8 = jnp.zeros((bn, 8), jnp.float32)
    z80 = jnp.zeros((bn, 80), jnp.float32)
    z112 = jnp.zeros((bn, 112), jnp.float32)
    t2a_ref[...] = jnp.concatenate([a2s, z8, h2[:, 0:112]], axis=1)
    t2b_ref[...] = jnp.concatenate([a2s, z8, h2[:, 112:224]], axis=1)
    t2c_ref[...] = jnp.concatenate([a2s, z8, h2[:, 224:256], z80], axis=1)
    ad2_ref[...] = jnp.concatenate([a2d, z8, z112], axis=1)

  return pl.pallas_call(
      body,
      grid=(grid,),
      in_specs=[
          pl.BlockSpec((2, bn, W), lambda i: (0, i, 0)),
          pl.BlockSpec((bn, W), lambda i: (i, 0)),
          pl.BlockSpec((bn, W), lambda i: (i, 0)),
          pl.BlockSpec((1, 64), lambda i: (0, 0)),
          pl.BlockSpec((64, 256), lambda i: (0, 0)),
          pl.BlockSpec((8, 32), lambda i: (0, 0)),
          pl.BlockSpec((8, 32), lambda i: (0, 0)),
      ],
      out_specs=[
          pl.BlockSpec((bn, W), lambda i: (i, 0)),
          pl.BlockSpec((bn, W), lambda i: (i, 0)),
          pl.BlockSpec((bn, W), lambda i: (i, 0)),
          pl.BlockSpec((bn, W), lambda i: (i, 0)),
      ],
      out_shape=[
          jax.ShapeDtypeStruct((n, W), jnp.float32),
          jax.ShapeDtypeStruct((n, W), jnp.float32),
          jax.ShapeDtypeStruct((n, W), jnp.float32),
          jax.ShapeDtypeStruct((n, W), jnp.float32),
      ],
  )(p1, ta1, adt1, b1, W2, as2, ad2)


def _mid2(p2a, p2b, p2c, t2a, t2b, t2c, ad2, b2, W3, as3, ad3, *, bn, grid):
  n = t2a.shape[0]

  def body(pa_ref, pb_ref, pc_ref, taa_ref, tab_ref, tac_ref, ad_ref,
           b_ref, w_ref, as_ref, adr_ref, ta3_ref, ad3_ref):
    numa = pa_ref[0, :, 16:128] + pa_ref[1, :, 16:128]
    numb = pb_ref[0, :, 16:128] + pb_ref[1, :, 16:128]
    numc = pc_ref[0, :, 16:48] + pc_ref[1, :, 16:48]
    den = pa_ref[0, :, 0:8] + pa_ref[1, :, 0:8]
    asrc = taa_ref[:, 0:8]
    adst = ad_ref[:, 0:8]
    h2 = jnp.concatenate(
        [taa_ref[:, 16:128], tab_ref[:, 16:128], tac_ref[:, 16:48]], axis=1)
    ws = jnp.exp(jnp.minimum(_lrelu(asrc + adst), CLAMP))
    den = den + ws
    num = jnp.concatenate([numa, numb, numc], axis=1)
    num = num + (ws[:, :, None] * h2.reshape(bn, 8, 32)).reshape(bn, 256)
    o = (num.reshape(bn, 8, 32) / den[:, :, None]).mean(axis=1)
    x3 = _elu(o + b_ref[...])
    h3 = jnp.dot(x3, w_ref[...], preferred_element_type=jnp.float32)
    a3s = (h3 * as_ref[...]).sum(-1, keepdims=True)
    a3d = (h3 * adr_ref[...]).sum(-1, keepdims=True)
    z96 = jnp.zeros((bn, 96), jnp.float32)
    z112 = jnp.zeros((bn, 112), jnp.float32)
    ta3_ref[...] = jnp.concatenate(
        [jnp.broadcast_to(a3s, (bn, 16)), h3, z96], axis=1)
    ad3_ref[...] = jnp.concatenate(
        [jnp.broadcast_to(a3d, (bn, 16)), z112], axis=1)

  return pl.pallas_call(
      body,
      grid=(grid,),
      in_specs=[
          pl.BlockSpec((2, bn, W), lambda i: (0, i, 0)),
          pl.BlockSpec((2, bn, W), lambda i: (0, i, 0)),
          pl.BlockSpec((2, bn, W), lambda i: (0, i, 0)),
          pl.BlockSpec((bn, W), lambda i: (i, 0)),
          pl.BlockSpec((bn, W), lambda i: (i, 0)),
          pl.BlockSpec((bn, W), lambda i: (i, 0)),
          pl.BlockSpec((bn, W), lambda i: (i, 0)),
          pl.BlockSpec((1, 32), lambda i: (0, 0)),
          pl.BlockSpec((32, 16), lambda i: (0, 0)),
          pl.BlockSpec((1, 16), lambda i: (0, 0)),
          pl.BlockSpec((1, 16), lambda i: (0, 0)),
      ],
      out_specs=[
          pl.BlockSpec((bn, W), lambda i: (i, 0)),
          pl.BlockSpec((bn, W), lambda i: (i, 0)),
      ],
      out_shape=[
          jax.ShapeDtypeStruct((n, W), jnp.float32),
          jax.ShapeDtypeStruct((n, W), jnp.float32),
      ],
  )(p2a, p2b, p2c, t2a, t2b, t2c, ad2, b2, W3, as3, ad3)


def _final(p3, ta3, adt3, b3, batch2d, linW, linb, *, n, ng):

  def body(p_ref, ta_ref, ad_ref, b_ref, bat_ref, lw_ref, lb_ref, o_ref):
    num = p_ref[0, :n, 16:32] + p_ref[1, :n, 16:32]
    den = p_ref[0, :n, 0:1] + p_ref[1, :n, 0:1]
    a = ta_ref[:, 0:1] + ad_ref[:, 0:1]
    ws = jnp.exp(jnp.minimum(_lrelu(a), CLAMP))
    den = den + ws
    num = num + ws * ta_ref[:, 16:32]
    o3 = num / den + b_ref[...]
    gids = lax.broadcasted_iota(jnp.int32, (ng, n), 0)
    onehot = (bat_ref[...] == gids).astype(jnp.float32)
    sums = jnp.dot(onehot, o3, preferred_element_type=jnp.float32)
    cnts = onehot.sum(axis=1, keepdims=True)
    pooled = sums / jnp.maximum(cnts, 1.0)
    o_ref[...] = (
        jnp.dot(pooled, lw_ref[...], preferred_element_type=jnp.float32)
        + lb_ref[...])

  return pl.pallas_call(
      body,
      out_shape=jax.ShapeDtypeStruct((ng, 2), jnp.float32),
  )(p3, ta3, adt3, b3, batch2d, linW, linb)


def _perm_l1():
  return np.concatenate(
      [np.repeat([2 * k, 2 * k + 1], 8) for k in range(4)]).astype(np.int32)


def _perm_l2(off, nh):
  return np.concatenate(
      [np.full(16, (off + 16 * k) // 32) for k in range(nh)]).astype(np.int32)


_PERM1 = _perm_l1()
_PERM2A = _perm_l2(0, 7)
_PERM2B = _perm_l2(112, 7)
_PERM2C = _perm_l2(224, 2)
_PERM3 = np.zeros(16, np.int32)
_DM8 = np.concatenate([np.ones(8), np.zeros(8)]).astype(np.float32)
_DM1 = np.concatenate([np.ones(1), np.zeros(15)]).astype(np.float32)
_DM0 = np.zeros(16, np.float32)


def kernel(x, edge_index, batch, W1, att_src1, att_dst1, b1,
           W2, att_src2, att_dst2, b2, W3, att_src3, att_dst3, b3,
           linW, linb):
  n = x.shape[0]
  e = edge_index.shape[1]
  ng = 32

  src = edge_index[0].astype(jnp.int32)
  dst = edge_index[1].astype(jnp.int32)
  nacc = (n + NS * ZCH) // (NS * ZCH) * (NS * ZCH)
  e_pad = -(-e // (NW * CH)) * (NW * CH)
  if e_pad > e:
    npad = e_pad - e
    spread = np.arange(npad, dtype=np.int32) % (nacc - n)
    src = jnp.concatenate([src, jnp.zeros((npad,), jnp.int32)])
    dst = jnp.concatenate([dst, jnp.asarray(spread + n, jnp.int32)])

  bn = n if n <= 2000 else 2000
  grid = n // bn

  as1 = att_src1.reshape(8, 8)
  ad1 = att_dst1.reshape(8, 8)
  as2 = att_src2.reshape(8, 32)
  ad2 = att_dst2.reshape(8, 32)
  as3 = att_src3.reshape(1, 16)
  ad3 = att_dst3.reshape(1, 16)
  b1r = b1.reshape(1, 64)
  b2r = b2.reshape(1, 32)
  b3r = b3.reshape(1, 16)
  lbr = linb.reshape(1, 2)
  batch2d = batch.astype(jnp.int32).reshape(1, n)

  zrows = jnp.zeros((ZCH, W), jnp.float32)
  perm1 = jnp.asarray(_PERM1)
  perm2a = jnp.asarray(_PERM2A)
  perm2b = jnp.asarray(_PERM2B)
  perm2c = jnp.asarray(_PERM2C)
  perm3 = jnp.asarray(_PERM3)
  dm8 = jnp.asarray(_DM8)
  dm1 = jnp.asarray(_DM1)
  dm0 = jnp.asarray(_DM0)

  ta1, adt1 = _prep1(x, W1, as1, ad1, bn=bn, grid=grid)
  p1 = _edge_pass(ta1, adt1, src, dst, zrows, perm1, dm8)
  t2a, t2b, t2c, adt2 = _mid1(p1, ta1, adt1, b1r, W2, as2, ad2,
                              bn=bn, grid=grid)
  p2a = _edge_pass(t2a, adt2, src, dst, zrows, perm2a, dm8)
  p2b = _edge_pass(t2b, adt2, src, dst, zrows, perm2b, dm0)
  p2c = _edge_pass(t2c, adt2, src, dst, zrows, perm2c, dm0)
  ta3, adt3 = _mid2(p2a, p2b, p2c, t2a, t2b, t2c, adt2, b2r, W3, as3, ad3,
                    bn=bn, grid=grid)
  p3 = _edge_pass(ta3, adt3, src, dst, zrows, perm3, dm1)
  return _final(p3, ta3, adt3, b3r, batch2d, linW, lbr, n=n, ng=ng)
